# baseline (speedup 1.0000x reference)
import functools

import jax
import jax.numpy as jnp
from jax import lax
from jax.experimental import pallas as pl
from jax.experimental.pallas import tpu as pltpu
from jax.experimental.pallas import tpu_sc as plsc

_N_ATOMS = 100000
_N_SEL = 50000
_D = 128
_DC = 3
_DCP = 16
_CHUNK = 80
_NCHUNK = _N_SEL // _CHUNK
_NW = 32
_ITERS = (_NCHUNK + _NW - 1) // _NW

_mesh = plsc.VectorSubcoreMesh(core_axis_name="c", subcore_axis_name="s")


@functools.partial(
    pl.kernel,
    mesh=_mesh,
    out_type=jax.ShapeDtypeStruct((_N_SEL, _D), jnp.float32),
    scratch_types=[
        pltpu.VMEM((_CHUNK,), jnp.int32),
        pltpu.VMEM((_CHUNK, _D), jnp.float32),
        pltpu.SemaphoreType.DMA,
    ],
)
def _gather_atm(idx_hbm, atm_hbm, atm_out, idx_v, rows_v, sem):
    w = lax.axis_index("s") * 2 + lax.axis_index("c")

    def body(i, carry):
        c = w + i * _NW

        @pl.when(c < _NCHUNK)
        def _():
            base = c * _CHUNK
            pltpu.sync_copy(idx_hbm.at[pl.ds(base, _CHUNK)], idx_v)
            pltpu.async_copy(atm_hbm.at[idx_v], rows_v, sem).wait()
            pltpu.sync_copy(rows_v, atm_out.at[pl.ds(base, _CHUNK)])

        return carry

    lax.fori_loop(0, _ITERS, body, 0)


@functools.partial(
    pl.kernel,
    mesh=_mesh,
    out_type=jax.ShapeDtypeStruct((_N_SEL, _DCP), jnp.float32),
    scratch_types=[
        pltpu.VMEM((_CHUNK,), jnp.int32),
        pltpu.VMEM((_CHUNK, _DCP), jnp.float32),
        pltpu.SemaphoreType.DMA,
    ],
    compiler_params=pltpu.CompilerParams(use_tc_tiling_on_sc=False),
)
def _gather_coord(idx_hbm, coord_hbm, coord_out, idx_v, crows_v, sem):
    w = lax.axis_index("s") * 2 + lax.axis_index("c")

    def body(i, carry):
        c = w + i * _NW

        @pl.when(c < _NCHUNK)
        def _():
            base = c * _CHUNK
            pltpu.sync_copy(idx_hbm.at[pl.ds(base, _CHUNK)], idx_v)
            pltpu.async_copy(coord_hbm.at[idx_v], crows_v, sem).wait()
            pltpu.sync_copy(crows_v, coord_out.at[pl.ds(base, _CHUNK)])

        return carry

    lax.fori_loop(0, _ITERS, body, 0)


def kernel(ent, atm, coord, mol_feat):
    e = jnp.reshape(ent, (_N_SEL,)).astype(jnp.int32)
    atm2 = jnp.reshape(atm, (_N_ATOMS, _D))
    coord2 = jnp.pad(jnp.reshape(coord, (_N_ATOMS, _DC)),
                     ((0, 0), (0, _DCP - _DC)))
    atm_sel = _gather_atm(e, atm2)
    coord_sel = _gather_coord(e, coord2)[:, :_DC]
    return (atm_sel[None], coord_sel[None], mol_feat)
